# baseline (speedup 1.0000x reference)
import functools

import jax
import jax.numpy as jnp
from jax import lax
from jax.experimental import pallas as pl
from jax.experimental.pallas import tpu as pltpu
from jax.experimental.pallas import tpu_sc as plsc

_N = 10000
_E = 320000
_D = 128
_H = 128
_O = 64
_G = 16

_NP = 10240
_NC = 2
_NS = 16
_EPT = _E // (_NC * _NS)
_CHUNK = 80
_NCHUNK = _EPT // _CHUNK
_NBUF = 5
_RPT = _NP // _NS
_ZROWS = 160

_HIGH = lax.Precision.HIGHEST


def _sc_degree(dst, zeros):
    import dataclasses
    mesh = plsc.VectorSubcoreMesh(core_axis_name="c", subcore_axis_name="s")
    cp = pltpu.CompilerParams()
    if "use_tc_tiling_on_sc" in pltpu.CompilerParams.__dataclass_fields__:
        cp = dataclasses.replace(cp, use_tc_tiling_on_sc=False)

    @functools.partial(
        pl.kernel,
        out_type=jax.ShapeDtypeStruct((_NC, _NP, 16), jnp.float32),
        mesh=mesh,
        compiler_params=cp,
        scratch_types=[
            pltpu.VMEM((_NCHUNK, _CHUNK), jnp.int32),
            pltpu.VMEM((_CHUNK, 16), jnp.float32),
            pltpu.VMEM_SHARED((_NP, 16), jnp.float32),
            pltpu.SemaphoreType.DMA,
        ],
    )
    def k(dst_hbm, zeros_hbm, out_hbm, dbuf, ones, acc, ssem):
        c = lax.axis_index("c")
        s = lax.axis_index("s")
        one16 = jnp.full((16,), 1.0, jnp.float32)

        zcopy = pltpu.async_copy(zeros_hbm.at[:, pl.ds(0, 16)],
                                 acc.at[pl.ds(s * _RPT, _RPT)], ssem)

        @pl.loop(0, _CHUNK)
        def _(i):
            ones[i, pl.ds(0, 16)] = one16

        tile = c * _NS + s
        pltpu.sync_copy(dst_hbm.at[tile], dbuf)
        zcopy.wait()
        plsc.subcore_barrier()

        @pl.loop(0, _NCHUNK // _NBUF)
        def _(grp):
            for b in range(_NBUF):
                pltpu.async_copy(ones, acc.at[dbuf.at[grp * _NBUF + b]],
                                 ssem, add=True)
            for b in range(_NBUF):
                pltpu.make_async_copy(ones, acc.at[dbuf.at[grp * _NBUF + b]],
                                      ssem).wait()

        plsc.subcore_barrier()
        pltpu.sync_copy(acc.at[pl.ds(s * _RPT, _RPT)],
                        out_hbm.at[c, pl.ds(s * _RPT, _RPT)])

    return k(dst, zeros)


def _sc_aggregate(g, src, dst, zeros):
    mesh = plsc.VectorSubcoreMesh(core_axis_name="c", subcore_axis_name="s")
    RING = 3
    ISLOTS = 2 * RING
    BODY = ISLOTS
    NIT = 19
    EPI = _NCHUNK - NIT * BODY

    @functools.partial(
        pl.kernel,
        out_type=jax.ShapeDtypeStruct((_NC, _NP, _H), jnp.float32),
        mesh=mesh,
        scratch_types=(
            [pltpu.VMEM((_CHUNK,), jnp.int32) for _ in range(2 * ISLOTS)]
            + [pltpu.VMEM((_CHUNK, _H), jnp.float32) for _ in range(RING)]
            + [pltpu.VMEM_SHARED((_NP, _H), jnp.float32)]
            + [pltpu.SemaphoreType.DMA for _ in range(ISLOTS + 2 * RING + 1)]
        ),
    )
    def k(g_hbm, src_hbm, dst_hbm, zeros_hbm, out_hbm, *rest):
        sidx = rest[:ISLOTS]
        didx = rest[ISLOTS:2 * ISLOTS]
        rows = rest[2 * ISLOTS:2 * ISLOTS + RING]
        acc = rest[2 * ISLOTS + RING]
        sems = rest[2 * ISLOTS + RING + 1:]
        isem = sems[:ISLOTS]
        gsem = sems[ISLOTS:ISLOTS + RING]
        ssem = sems[ISLOTS + RING:ISLOTS + 2 * RING]
        zsem = sems[ISLOTS + 2 * RING]
        c = lax.axis_index("c")
        s = lax.axis_index("s")

        zcopy = pltpu.async_copy(zeros_hbm, acc.at[pl.ds(s * _RPT, _RPT)],
                                 zsem)

        base = (c * _NS + s) * _EPT

        def start_idx(j, i6):
            off = base + j * _CHUNK
            pltpu.async_copy(src_hbm.at[pl.ds(off, _CHUNK)], sidx[i6],
                             isem[i6])
            pltpu.async_copy(dst_hbm.at[pl.ds(off, _CHUNK)], didx[i6],
                             isem[i6])

        def wait_idx(j, i6):
            off = base + j * _CHUNK
            pltpu.make_async_copy(src_hbm.at[pl.ds(off, _CHUNK)], sidx[i6],
                                  isem[i6]).wait()
            pltpu.make_async_copy(dst_hbm.at[pl.ds(off, _CHUNK)], didx[i6],
                                  isem[i6]).wait()

        def start_gather(i6, i3):
            pltpu.async_copy(g_hbm.at[sidx[i6]], rows[i3], gsem[i3])

        def wait_gather(i6, i3):
            pltpu.make_async_copy(g_hbm.at[sidx[i6]], rows[i3],
                                  gsem[i3]).wait()

        def start_scatter(i6, i3):
            pltpu.async_copy(rows[i3], acc.at[didx[i6]], ssem[i3],
                             add=True)

        def wait_scatter(i6, i3):
            pltpu.make_async_copy(rows[i3], acc.at[didx[i6]],
                                  ssem[i3]).wait()

        def chunk_step(j, u, first, last_idx, last_gather):
            wait_gather(u, u % RING)
            start_scatter(u, u % RING)
            if not first:
                wait_scatter((u - 1) % ISLOTS, (u - 1) % RING)
            if last_idx:
                start_idx(j + ISLOTS - 1, (u + ISLOTS - 1) % ISLOTS)
            if last_gather:
                wait_idx(j + 2, (u + 2) % ISLOTS)
                start_gather((u + 2) % ISLOTS, (u + 2) % RING)

        for b in range(ISLOTS - 1):
            start_idx(b, b)
        for b in range(2):
            wait_idx(b, b)
            start_gather(b, b)
        zcopy.wait()
        plsc.subcore_barrier()

        for j in range(BODY):
            chunk_step(j, j, j == 0, True, True)

        @pl.loop(1, NIT)
        def _(it):
            j0 = it * BODY
            for u in range(BODY):
                chunk_step(j0 + u, u, False, True, True)

        for j in range(NIT * BODY, _NCHUNK):
            u = j % ISLOTS
            chunk_step(j, u, False,
                       j + ISLOTS - 1 < _NCHUNK, j + 2 < _NCHUNK)
        wait_scatter((_NCHUNK - 1) % ISLOTS, (_NCHUNK - 1) % RING)

        plsc.subcore_barrier()
        pltpu.sync_copy(acc.at[pl.ds(s * _RPT, _RPT)],
                        out_hbm.at[c, pl.ds(s * _RPT, _RPT)])

    return k(g, src, dst, zeros)


_BLK = 2000
_NBLK = _N // _BLK


def _dinv_from(dp_ref):
    return lax.rsqrt(dp_ref[0, :, 0:1] + dp_ref[1, :, 0:1] + 1.0)


def _tc_g1(x, w1, dp):
    def body(x_ref, dp_ref, w_ref, o_ref):
        xw = jnp.dot(x_ref[...], w_ref[...],
                     preferred_element_type=jnp.float32,
                     precision=_HIGH)
        o_ref[...] = xw * _dinv_from(dp_ref)

    return pl.pallas_call(
        body,
        grid=(_NBLK,),
        in_specs=[
            pl.BlockSpec((_BLK, _D), lambda i: (i, 0)),
            pl.BlockSpec((_NC, _BLK, 16), lambda i: (0, i, 0)),
            pl.BlockSpec((_D, _H), lambda i: (0, 0)),
        ],
        out_specs=pl.BlockSpec((_BLK, _H), lambda i: (i, 0)),
        out_shape=jax.ShapeDtypeStruct((_N, _H), jnp.float32),
    )(x, dp, w1)


def _tc_layer2(a1, g1, dp, b1, w2):
    def body(a_ref, g_ref, dp_ref, b_ref, w_ref, o_ref):
        dinv = _dinv_from(dp_ref)
        h = jnp.maximum(dinv * (a_ref[0] + a_ref[1] + g_ref[...]) + b_ref[...],
                        0.0)
        o_ref[...] = jnp.dot(h, w_ref[...],
                             preferred_element_type=jnp.float32,
                             precision=_HIGH) * dinv

    return pl.pallas_call(
        body,
        grid=(_NBLK,),
        in_specs=[
            pl.BlockSpec((_NC, _BLK, _H), lambda i: (0, i, 0)),
            pl.BlockSpec((_BLK, _H), lambda i: (i, 0)),
            pl.BlockSpec((_NC, _BLK, 16), lambda i: (0, i, 0)),
            pl.BlockSpec((1, _H), lambda i: (0, 0)),
            pl.BlockSpec((_H, _H), lambda i: (0, 0)),
        ],
        out_specs=pl.BlockSpec((_BLK, _H), lambda i: (i, 0)),
        out_shape=jax.ShapeDtypeStruct((_N, _H), jnp.float32),
    )(a1, g1, dp, b1, w2)


def _tc_head(a2, g2, dp, b2, batch3, wfc, bfc):
    def body(a_ref, g_ref, dp_ref, b_ref, bat_ref, wfc_ref, bfc_ref, o_ref,
             sum_ref, cnt_ref):
        i = pl.program_id(0)

        @pl.when(i == 0)
        def _():
            sum_ref[...] = jnp.zeros((_G, _H), jnp.float32)
            cnt_ref[...] = jnp.zeros((_G, 128), jnp.float32)

        dinv = _dinv_from(dp_ref)
        h = jnp.maximum(dinv * (a_ref[0] + a_ref[1] + g_ref[...]) + b_ref[...],
                        0.0)
        gid = lax.broadcasted_iota(jnp.int32, (_G, _BLK), 0)
        mask = (bat_ref[0] == gid).astype(jnp.float32)
        sum_ref[...] += jnp.dot(mask, h, preferred_element_type=jnp.float32,
                                precision=_HIGH)
        cnt_ref[...] += jnp.sum(mask, axis=1, keepdims=True)

        @pl.when(i == _NBLK - 1)
        def _():
            pooled = sum_ref[...] / jnp.maximum(cnt_ref[:, 0:1], 1.0)
            logits = jnp.dot(pooled, wfc_ref[...],
                             preferred_element_type=jnp.float32,
                             precision=_HIGH) + bfc_ref[...]
            shifted = logits - jnp.max(logits, axis=-1, keepdims=True)
            o_ref[...] = shifted - jnp.log(
                jnp.sum(jnp.exp(shifted), axis=-1, keepdims=True))

    return pl.pallas_call(
        body,
        grid=(_NBLK,),
        in_specs=[
            pl.BlockSpec((_NC, _BLK, _H), lambda i: (0, i, 0)),
            pl.BlockSpec((_BLK, _H), lambda i: (i, 0)),
            pl.BlockSpec((_NC, _BLK, 16), lambda i: (0, i, 0)),
            pl.BlockSpec((1, _H), lambda i: (0, 0)),
            pl.BlockSpec((1, 1, _BLK), lambda i: (i, 0, 0)),
            pl.BlockSpec((_H, _O), lambda i: (0, 0)),
            pl.BlockSpec((1, _O), lambda i: (0, 0)),
        ],
        out_specs=pl.BlockSpec((_G, _O), lambda i: (0, 0)),
        out_shape=jax.ShapeDtypeStruct((_G, _O), jnp.float32),
        scratch_shapes=[
            pltpu.VMEM((_G, _H), jnp.float32),
            pltpu.VMEM((_G, 128), jnp.float32),
        ],
    )(a2, g2, dp, b2, batch3, wfc, bfc)


def kernel(x, edge_index, batch, W1, b1, W2, b2, Wfc, bfc):
    src = edge_index[0]
    dst = edge_index[1]
    dst3 = dst.reshape(_NC * _NS, _NCHUNK, _CHUNK)

    zeros = jnp.zeros((_RPT, _H), jnp.float32)

    dp = _sc_degree(dst3, zeros)
    g1 = _tc_g1(x, W1, dp)
    a1 = _sc_aggregate(g1, src, dst, zeros)
    g2 = _tc_layer2(a1, g1, dp, b1.reshape(1, _H), W2)
    a2 = _sc_aggregate(g2, src, dst, zeros)
    return _tc_head(a2, g2, dp, b2.reshape(1, _H),
                    batch.reshape(_NBLK, 1, _BLK), Wfc, bfc.reshape(1, _O))


# speedup vs baseline: 1.0326x; 1.0326x over previous
import functools

import jax
import jax.numpy as jnp
from jax import lax
from jax.experimental import pallas as pl
from jax.experimental.pallas import tpu as pltpu
from jax.experimental.pallas import tpu_sc as plsc

_N = 10000
_E = 320000
_D = 128
_H = 128
_O = 64
_G = 16

_NP = 10240
_NC = 2
_NS = 16
_EPT = _E // (_NC * _NS)
_CHUNK = 80
_NCHUNK = _EPT // _CHUNK
_NBUF = 5
_RPT = _NP // _NS
_ZROWS = 160

_HIGH = lax.Precision.HIGHEST


def _sc_degree(dst, zeros):
    import dataclasses
    mesh = plsc.VectorSubcoreMesh(core_axis_name="c", subcore_axis_name="s")
    cp = pltpu.CompilerParams()
    if "use_tc_tiling_on_sc" in pltpu.CompilerParams.__dataclass_fields__:
        cp = dataclasses.replace(cp, use_tc_tiling_on_sc=False)

    @functools.partial(
        pl.kernel,
        out_type=jax.ShapeDtypeStruct((_NC, _NP, 16), jnp.float32),
        mesh=mesh,
        compiler_params=cp,
        scratch_types=[
            pltpu.VMEM((_NCHUNK, _CHUNK), jnp.int32),
            pltpu.VMEM((_CHUNK, 16), jnp.float32),
            pltpu.VMEM_SHARED((_NP, 16), jnp.float32),
            pltpu.SemaphoreType.DMA,
        ],
    )
    def k(dst_hbm, zeros_hbm, out_hbm, dbuf, ones, acc, ssem):
        c = lax.axis_index("c")
        s = lax.axis_index("s")
        one16 = jnp.full((16,), 1.0, jnp.float32)

        zcopy = pltpu.async_copy(zeros_hbm.at[:, pl.ds(0, 16)],
                                 acc.at[pl.ds(s * _RPT, _RPT)], ssem)

        @pl.loop(0, _CHUNK)
        def _(i):
            ones[i, pl.ds(0, 16)] = one16

        tile = c * _NS + s
        pltpu.sync_copy(dst_hbm.at[tile], dbuf)
        zcopy.wait()
        plsc.subcore_barrier()

        @pl.loop(0, _NCHUNK // _NBUF)
        def _(grp):
            for b in range(_NBUF):
                pltpu.async_copy(ones, acc.at[dbuf.at[grp * _NBUF + b]],
                                 ssem, add=True)
            for b in range(_NBUF):
                pltpu.make_async_copy(ones, acc.at[dbuf.at[grp * _NBUF + b]],
                                      ssem).wait()

        plsc.subcore_barrier()
        pltpu.sync_copy(acc.at[pl.ds(s * _RPT, _RPT)],
                        out_hbm.at[c, pl.ds(s * _RPT, _RPT)])

    return k(dst, zeros)


def _sc_aggregate(g, src, dst, zeros):
    mesh = plsc.VectorSubcoreMesh(core_axis_name="c", subcore_axis_name="s")
    RING = 3
    ISLOTS = 2 * RING
    BODY = ISLOTS
    NIT = 19
    EPI = _NCHUNK - NIT * BODY

    @functools.partial(
        pl.kernel,
        out_type=jax.ShapeDtypeStruct((_NC, _NP, _H), jnp.float32),
        mesh=mesh,
        scratch_types=(
            [pltpu.VMEM((_CHUNK,), jnp.int32) for _ in range(2 * ISLOTS)]
            + [pltpu.VMEM((_CHUNK, _H), jnp.float32) for _ in range(RING)]
            + [pltpu.VMEM_SHARED((_NP, _H), jnp.float32)]
            + [pltpu.SemaphoreType.DMA for _ in range(ISLOTS + RING + 1)]
        ),
    )
    def k(g_hbm, src_hbm, dst_hbm, zeros_hbm, out_hbm, *rest):
        sidx = rest[:ISLOTS]
        didx = rest[ISLOTS:2 * ISLOTS]
        rows = rest[2 * ISLOTS:2 * ISLOTS + RING]
        acc = rest[2 * ISLOTS + RING]
        isem = rest[2 * ISLOTS + RING + 1:2 * ISLOTS + RING + 1 + ISLOTS]
        gsem = rest[2 * ISLOTS + RING + 1 + ISLOTS:
                    2 * ISLOTS + RING + 1 + ISLOTS + RING]
        zsem = rest[2 * ISLOTS + RING + 1 + ISLOTS + RING]
        c = lax.axis_index("c")
        s = lax.axis_index("s")

        zcopy = pltpu.async_copy(zeros_hbm, acc.at[pl.ds(s * _RPT, _RPT)],
                                 zsem)

        base = (c * _NS + s) * _EPT

        def start_idx(j, i6):
            off = base + j * _CHUNK
            pltpu.async_copy(src_hbm.at[pl.ds(off, _CHUNK)], sidx[i6],
                             isem[i6])
            pltpu.async_copy(dst_hbm.at[pl.ds(off, _CHUNK)], didx[i6],
                             isem[i6])

        def wait_idx(j, i6):
            off = base + j * _CHUNK
            pltpu.make_async_copy(src_hbm.at[pl.ds(off, _CHUNK)], sidx[i6],
                                  isem[i6]).wait()
            pltpu.make_async_copy(dst_hbm.at[pl.ds(off, _CHUNK)], didx[i6],
                                  isem[i6]).wait()

        def start_gather(i6, i3):
            pltpu.async_copy(g_hbm.at[sidx[i6]], rows[i3], gsem[i3])

        def finish_chunk(i6, i3):
            pltpu.make_async_copy(g_hbm.at[sidx[i6]], rows[i3],
                                  gsem[i3]).wait()
            pltpu.sync_copy(rows[i3], acc.at[didx[i6]], add=True)

        for b in range(ISLOTS):
            start_idx(b, b)
        for b in range(RING):
            wait_idx(b, b)
            start_gather(b, b)
        zcopy.wait()
        plsc.subcore_barrier()

        @pl.loop(0, NIT)
        def _(it):
            j0 = it * BODY
            for u in range(BODY):
                j = j0 + u
                finish_chunk(u, u % RING)
                start_idx(j + ISLOTS, u)
                wait_idx(j + RING, (u + RING) % ISLOTS)
                start_gather((u + RING) % ISLOTS, u % RING)

        for j in range(NIT * BODY, _NCHUNK):
            u = j % ISLOTS
            finish_chunk(u, u % RING)
            if j + ISLOTS < _NCHUNK:
                start_idx(j + ISLOTS, u)
            if j + RING < _NCHUNK:
                wait_idx(j + RING, (u + RING) % ISLOTS)
                start_gather((u + RING) % ISLOTS, u % RING)

        plsc.subcore_barrier()
        pltpu.sync_copy(acc.at[pl.ds(s * _RPT, _RPT)],
                        out_hbm.at[c, pl.ds(s * _RPT, _RPT)])

    return k(g, src, dst, zeros)


_BLK = 2000
_NBLK = _N // _BLK


def _dinv_from(dp_ref):
    return lax.rsqrt(dp_ref[0, :, 0:1] + dp_ref[1, :, 0:1] + 1.0)


def _tc_g1(x, w1, dp):
    def body(x_ref, dp_ref, w_ref, o_ref):
        xw = jnp.dot(x_ref[...], w_ref[...],
                     preferred_element_type=jnp.float32,
                     precision=_HIGH)
        o_ref[...] = xw * _dinv_from(dp_ref)

    return pl.pallas_call(
        body,
        grid=(_NBLK,),
        in_specs=[
            pl.BlockSpec((_BLK, _D), lambda i: (i, 0)),
            pl.BlockSpec((_NC, _BLK, 16), lambda i: (0, i, 0)),
            pl.BlockSpec((_D, _H), lambda i: (0, 0)),
        ],
        out_specs=pl.BlockSpec((_BLK, _H), lambda i: (i, 0)),
        out_shape=jax.ShapeDtypeStruct((_N, _H), jnp.float32),
    )(x, dp, w1)


def _tc_layer2(a1, g1, dp, b1, w2):
    def body(a_ref, g_ref, dp_ref, b_ref, w_ref, o_ref):
        dinv = _dinv_from(dp_ref)
        h = jnp.maximum(dinv * (a_ref[0] + a_ref[1] + g_ref[...]) + b_ref[...],
                        0.0)
        o_ref[...] = jnp.dot(h, w_ref[...],
                             preferred_element_type=jnp.float32,
                             precision=_HIGH) * dinv

    return pl.pallas_call(
        body,
        grid=(_NBLK,),
        in_specs=[
            pl.BlockSpec((_NC, _BLK, _H), lambda i: (0, i, 0)),
            pl.BlockSpec((_BLK, _H), lambda i: (i, 0)),
            pl.BlockSpec((_NC, _BLK, 16), lambda i: (0, i, 0)),
            pl.BlockSpec((1, _H), lambda i: (0, 0)),
            pl.BlockSpec((_H, _H), lambda i: (0, 0)),
        ],
        out_specs=pl.BlockSpec((_BLK, _H), lambda i: (i, 0)),
        out_shape=jax.ShapeDtypeStruct((_N, _H), jnp.float32),
    )(a1, g1, dp, b1, w2)


def _tc_head(a2, g2, dp, b2, batch3, wfc, bfc):
    def body(a_ref, g_ref, dp_ref, b_ref, bat_ref, wfc_ref, bfc_ref, o_ref,
             sum_ref, cnt_ref):
        i = pl.program_id(0)

        @pl.when(i == 0)
        def _():
            sum_ref[...] = jnp.zeros((_G, _H), jnp.float32)
            cnt_ref[...] = jnp.zeros((_G, 128), jnp.float32)

        dinv = _dinv_from(dp_ref)
        h = jnp.maximum(dinv * (a_ref[0] + a_ref[1] + g_ref[...]) + b_ref[...],
                        0.0)
        gid = lax.broadcasted_iota(jnp.int32, (_G, _BLK), 0)
        mask = (bat_ref[0] == gid).astype(jnp.float32)
        sum_ref[...] += jnp.dot(mask, h, preferred_element_type=jnp.float32,
                                precision=_HIGH)
        cnt_ref[...] += jnp.sum(mask, axis=1, keepdims=True)

        @pl.when(i == _NBLK - 1)
        def _():
            pooled = sum_ref[...] / jnp.maximum(cnt_ref[:, 0:1], 1.0)
            logits = jnp.dot(pooled, wfc_ref[...],
                             preferred_element_type=jnp.float32,
                             precision=_HIGH) + bfc_ref[...]
            shifted = logits - jnp.max(logits, axis=-1, keepdims=True)
            o_ref[...] = shifted - jnp.log(
                jnp.sum(jnp.exp(shifted), axis=-1, keepdims=True))

    return pl.pallas_call(
        body,
        grid=(_NBLK,),
        in_specs=[
            pl.BlockSpec((_NC, _BLK, _H), lambda i: (0, i, 0)),
            pl.BlockSpec((_BLK, _H), lambda i: (i, 0)),
            pl.BlockSpec((_NC, _BLK, 16), lambda i: (0, i, 0)),
            pl.BlockSpec((1, _H), lambda i: (0, 0)),
            pl.BlockSpec((1, 1, _BLK), lambda i: (i, 0, 0)),
            pl.BlockSpec((_H, _O), lambda i: (0, 0)),
            pl.BlockSpec((1, _O), lambda i: (0, 0)),
        ],
        out_specs=pl.BlockSpec((_G, _O), lambda i: (0, 0)),
        out_shape=jax.ShapeDtypeStruct((_G, _O), jnp.float32),
        scratch_shapes=[
            pltpu.VMEM((_G, _H), jnp.float32),
            pltpu.VMEM((_G, 128), jnp.float32),
        ],
    )(a2, g2, dp, b2, batch3, wfc, bfc)


def kernel(x, edge_index, batch, W1, b1, W2, b2, Wfc, bfc):
    src = edge_index[0]
    dst = edge_index[1]
    dst3 = dst.reshape(_NC * _NS, _NCHUNK, _CHUNK)

    zeros = jnp.zeros((_RPT, _H), jnp.float32)

    dp = _sc_degree(dst3, zeros)
    g1 = _tc_g1(x, W1, dp)
    a1 = _sc_aggregate(g1, src, dst, zeros)
    g2 = _tc_layer2(a1, g1, dp, b1.reshape(1, _H), W2)
    a2 = _sc_aggregate(g2, src, dst, zeros)
    return _tc_head(a2, g2, dp, b2.reshape(1, _H),
                    batch.reshape(_NBLK, 1, _BLK), Wfc, bfc.reshape(1, _O))
